# Initial kernel scaffold; baseline (speedup 1.0000x reference)
import jax
import jax.numpy as jnp
from jax.experimental import pallas as pl


def kernel(features, edge_index, W0, b0, W1, b1, W2, b2):
    raise NotImplementedError("write your pallas kernel here")


# speedup vs baseline: 7.7801x; 7.7801x over previous
import functools

import jax
import jax.numpy as jnp
from jax import lax
from jax.experimental import pallas as pl
from jax.experimental.pallas import tpu as pltpu
from jax.experimental.pallas import tpu_sc as plsc

NC = 2
NT = 16
LANE = 16
DH = 64
CW = 128
TR = 64


def _mesh():
    return plsc.VectorSubcoreMesh(
        core_axis_name="c", subcore_axis_name="s", num_cores=NC,
        num_subcores=NT)


def _wb_split(N):
    wbt = next(t for t in range(NT, 0, -1) if N % t == 0 and (N // t) % 8 == 0)
    return wbt, N // wbt


def _make_deg_kernel(N, EP):
    B = 4096
    R = B // CW
    ept = EP // NT
    nb = ept // B
    assert ept % B == 0
    slab = ((-(-N // NT) + LANE - 1) // LANE) * LANE
    npad = slab * NT
    assert npad >= N + TR
    wbt, wb = _wb_split(N)

    @functools.partial(
        pl.kernel,
        out_type=jax.ShapeDtypeStruct((2 * N,), jnp.float32),
        mesh=_mesh(),
        scratch_types=[
            pltpu.VMEM((R, CW), jnp.int32),
            pltpu.VMEM((slab,), jnp.float32),
            pltpu.VMEM((CW,), jnp.float32),
            pltpu.VMEM((wb,), jnp.float32),
            pltpu.VMEM_SHARED((npad,), jnp.float32),
        ],
    )
    def deg_kernel(ends_hbm, out_hbm, idx_m, zbuf, ones_v, dbuf, acc):
        c = lax.axis_index("c")
        s = lax.axis_index("s")
        zero16 = jnp.zeros((LANE,), jnp.float32)
        one16 = jnp.ones((LANE,), jnp.float32)
        for j in range(slab // LANE):
            zbuf[pl.ds(j * LANE, LANE)] = zero16
        for j in range(CW // LANE):
            ones_v[pl.ds(j * LANE, LANE)] = one16
        pltpu.sync_copy(zbuf, acc.at[pl.ds(s * slab, slab)])
        plsc.subcore_barrier()

        def body(i, carry):
            rbase = pl.multiple_of((c * EP + s * ept + i * B) // CW, 8)
            pltpu.sync_copy(ends_hbm.at[pl.ds(rbase, R)], idx_m)
            for k in range(R):
                pltpu.sync_copy(ones_v, acc.at[idx_m.at[k]], add=True)
            return carry

        lax.fori_loop(0, nb, body, 0)
        plsc.subcore_barrier()

        @pl.when(s < wbt)
        def _():
            pltpu.sync_copy(acc.at[pl.ds(pl.multiple_of(s * wb, 8), wb)],
                            dbuf)
            pltpu.sync_copy(
                dbuf, out_hbm.at[pl.ds(pl.multiple_of(c * N + s * wb, 8), wb)])

    return deg_kernel


def _make_agg_kernel(N, EP, S):
    ppc = S // NC
    B = 1024
    R = B // CW
    ept = EP // NT
    nb = ept // B
    assert ept % B == 0
    wbt, wb = _wb_split(N)
    zr = 40
    wzr = 200
    assert wb % zr == 0 and wb % wzr == 0 and B >= wzr

    @functools.partial(
        pl.kernel,
        out_type=jax.ShapeDtypeStruct((S * N, DH), jnp.float32),
        mesh=_mesh(),
        scratch_types=[
            pltpu.VMEM((R, CW), jnp.int32),
            pltpu.VMEM((R, CW), jnp.int32),
            pltpu.VMEM((B, DH), jnp.float32),
            pltpu.VMEM((zr, DH), jnp.float32),
            pltpu.VMEM_SHARED((N + TR, DH), jnp.float32),
            pltpu.SemaphoreType.DMA,
        ],
        compiler_params=pltpu.CompilerParams(use_tc_tiling_on_sc=False),
    )
    def agg_kernel(x_hbm, gidx_hbm, dst_hbm, out_hbm,
                   gi_m, si_m, rows_v, zbuf, acc, sem):
        c = lax.axis_index("c")
        s = lax.axis_index("s")
        zero16 = jnp.zeros((LANE,), jnp.float32)
        for r in range(zr):
            for j in range(DH // LANE):
                zbuf[r, pl.ds(j * LANE, LANE)] = zero16

        erows = EP // CW
        etrows = ept // CW

        for p in range(ppc):
            q = c * ppc + p

            @pl.when(s < wbt)
            def _():
                def zloop(k, carry):
                    base = pl.multiple_of(s * wb + k * zr, 8)
                    pltpu.sync_copy(zbuf, acc.at[pl.ds(base, zr)])
                    return carry

                lax.fori_loop(0, wb // zr, zloop, 0)

            plsc.subcore_barrier()

            def body(i, carry):
                rbase = pl.multiple_of(s * etrows + i * R, 8)
                pltpu.sync_copy(
                    gidx_hbm.at[pl.ds(pl.multiple_of(q * erows + rbase, 8),
                                      R)], gi_m)
                pltpu.sync_copy(dst_hbm.at[pl.ds(rbase, R)], si_m)
                gets = [
                    pltpu.async_copy(x_hbm.at[gi_m.at[k]],
                                     rows_v.at[pl.ds(k * CW, CW)], sem)
                    for k in range(R)
                ]
                for g in gets:
                    g.wait()
                for k in range(R):
                    pltpu.sync_copy(rows_v.at[pl.ds(k * CW, CW)],
                                    acc.at[si_m.at[k]], add=True)
                return carry

            lax.fori_loop(0, nb, body, 0)
            plsc.subcore_barrier()

            @pl.when(s < wbt)
            def _():
                def wloop(k, carry):
                    base = pl.multiple_of(s * wb + k * wzr, 8)
                    bb = rows_v.at[pl.ds(0, wzr)]
                    pltpu.sync_copy(acc.at[pl.ds(base, wzr)], bb)
                    obase = pl.multiple_of(q * N + base, 8)
                    pltpu.sync_copy(bb, out_hbm.at[pl.ds(obase, wzr)])
                    return carry

                lax.fori_loop(0, wb // wzr, wloop, 0)

            plsc.subcore_barrier()

    return agg_kernel


def _to_seg(x):
    return jnp.stack([x[:, q * DH:(q + 1) * DH]
                      for q in range(x.shape[-1] // DH)])


def _from_seg(a):
    return jnp.concatenate(list(a), axis=-1)


def _prep_body(deg_ref, feat_ref, inv_ref, x_ref):
    inv = lax.rsqrt(jnp.maximum(deg_ref[...], 1.0))
    inv_ref[...] = inv
    x_ref[...] = _to_seg(feat_ref[...] * inv[0])


def _l1_body(agg_ref, inv_ref, w_ref, b_ref, out_ref):
    ag = _from_seg(agg_ref[...])
    inv = inv_ref[...]
    h = jnp.dot(ag * inv[1], w_ref[...], preferred_element_type=jnp.float32)
    h = jnp.maximum(h + b_ref[...], 0.0)
    out_ref[...] = _to_seg(h * inv[0])


def _l2_body(agg_ref, inv_ref, w1_ref, b1_ref, w2_ref, out_ref):
    ag = _from_seg(agg_ref[...])
    inv = inv_ref[...]
    h = jnp.dot(ag * inv[1], w1_ref[...], preferred_element_type=jnp.float32)
    h = jnp.maximum(h + b1_ref[...], 0.0)
    z = jnp.dot(h * inv[0], w2_ref[...], preferred_element_type=jnp.float32)
    out_ref[...] = _to_seg(z)


def _l3_body(agg_ref, inv_ref, b_ref, out_ref):
    ag = _from_seg(agg_ref[...])
    out_ref[...] = ag * inv_ref[...][1] + b_ref[...]


def _row_spec(shape):
    if len(shape) == 3:
        return pl.BlockSpec(shape, lambda i: (0, i, 0))
    return pl.BlockSpec(shape, lambda i: (i, 0))


def _full_spec(shape):
    nd = len(shape)
    return pl.BlockSpec(shape, lambda i: (0,) * nd)


def kernel(features, edge_index, W0, b0, W1, b1, W2, b2):
    N, D = features.shape
    E = edge_index.shape[1]
    H1 = W0.shape[1]
    H2 = W1.shape[1]
    C = W2.shape[1]
    BN = 2000
    assert N % BN == 0 and D == 2 * DH and H1 == 4 * DH and C == 2 * DH
    grid = N // BN

    chunk = NT * 512
    EP = -(-E // chunk) * chunk
    npads = EP - E
    src = edge_index[0]
    dst = edge_index[1]
    spread = jnp.arange(npads, dtype=jnp.int32) % TR
    trash = N + spread
    dstp = jnp.concatenate([dst, trash])
    dst2d = dstp.reshape(EP // CW, CW)

    def gidx(S):
        parts = []
        for q in range(S):
            parts.append(src + q * N)
            parts.append(q * N + spread)
        return jnp.concatenate(parts).reshape(S * EP // CW, CW)

    gidx2 = gidx(2)
    gidx4 = gidx(4)
    ends2d = jnp.concatenate([src, trash, dst, trash]).reshape(
        2 * EP // CW, CW)

    deg = _make_deg_kernel(N, EP)(ends2d)
    degs = deg.reshape(2, N, 1)

    invs, x0 = pl.pallas_call(
        _prep_body,
        grid=(grid,),
        in_specs=[_row_spec((2, BN, 1)), _row_spec((BN, D))],
        out_specs=[_row_spec((2, BN, 1)), _row_spec((2, BN, DH))],
        out_shape=[jax.ShapeDtypeStruct((2, N, 1), jnp.float32),
                   jax.ShapeDtypeStruct((2, N, DH), jnp.float32)],
    )(degs, features)

    agg0 = _make_agg_kernel(N, EP, 2)(x0.reshape(2 * N, DH), gidx2, dst2d)

    x1 = pl.pallas_call(
        _l1_body,
        grid=(grid,),
        in_specs=[_row_spec((2, BN, DH)), _row_spec((2, BN, 1)),
                  _full_spec((D, H1)), _full_spec((1, H1))],
        out_specs=_row_spec((4, BN, DH)),
        out_shape=jax.ShapeDtypeStruct((4, N, DH), jnp.float32),
    )(agg0.reshape(2, N, DH), invs, W0, b0.reshape(1, H1))

    agg1 = _make_agg_kernel(N, EP, 4)(x1.reshape(4 * N, DH), gidx4, dst2d)

    z2 = pl.pallas_call(
        _l2_body,
        grid=(grid,),
        in_specs=[_row_spec((4, BN, DH)), _row_spec((2, BN, 1)),
                  _full_spec((H1, H2)), _full_spec((1, H2)),
                  _full_spec((H2, C))],
        out_specs=_row_spec((2, BN, DH)),
        out_shape=jax.ShapeDtypeStruct((2, N, DH), jnp.float32),
    )(agg1.reshape(4, N, DH), invs, W1, b1.reshape(1, H2), W2)

    agg2 = _make_agg_kernel(N, EP, 2)(z2.reshape(2 * N, DH), gidx2, dst2d)

    out = pl.pallas_call(
        _l3_body,
        grid=(grid,),
        in_specs=[_row_spec((2, BN, DH)), _row_spec((2, BN, 1)),
                  _full_spec((1, C))],
        out_specs=_row_spec((BN, C)),
        out_shape=jax.ShapeDtypeStruct((N, C), jnp.float32),
    )(agg2.reshape(2, N, DH), invs, b2.reshape(1, C))

    return out


# speedup vs baseline: 9.0659x; 1.1653x over previous
import functools

import jax
import jax.numpy as jnp
from jax import lax
from jax.experimental import pallas as pl
from jax.experimental.pallas import tpu as pltpu
from jax.experimental.pallas import tpu_sc as plsc

NC = 2
NT = 16
LANE = 16
DH = 64
CW = 128
TR = 64


def _mesh():
    return plsc.VectorSubcoreMesh(
        core_axis_name="c", subcore_axis_name="s", num_cores=NC,
        num_subcores=NT)


def _wb_split(N):
    wbt = next(t for t in range(NT, 0, -1) if N % t == 0 and (N // t) % 8 == 0)
    return wbt, N // wbt


def _make_deg_kernel(N, EP):
    B = 4096
    R = B // CW
    ept = EP // NT
    nb = ept // B
    assert ept % B == 0
    slab = ((-(-N // NT) + LANE - 1) // LANE) * LANE
    npad = slab * NT
    assert npad >= N + TR
    wbt, wb = _wb_split(N)

    @functools.partial(
        pl.kernel,
        out_type=jax.ShapeDtypeStruct((2 * N,), jnp.float32),
        mesh=_mesh(),
        scratch_types=[
            pltpu.VMEM((R, CW), jnp.int32),
            pltpu.VMEM((slab,), jnp.float32),
            pltpu.VMEM((CW,), jnp.float32),
            pltpu.VMEM((wb,), jnp.float32),
            pltpu.VMEM_SHARED((npad,), jnp.float32),
        ],
    )
    def deg_kernel(ends_hbm, out_hbm, idx_m, zbuf, ones_v, dbuf, acc):
        c = lax.axis_index("c")
        s = lax.axis_index("s")
        zero16 = jnp.zeros((LANE,), jnp.float32)
        one16 = jnp.ones((LANE,), jnp.float32)
        for j in range(slab // LANE):
            zbuf[pl.ds(j * LANE, LANE)] = zero16
        for j in range(CW // LANE):
            ones_v[pl.ds(j * LANE, LANE)] = one16
        pltpu.sync_copy(zbuf, acc.at[pl.ds(s * slab, slab)])
        plsc.subcore_barrier()

        def body(i, carry):
            rbase = pl.multiple_of((c * EP + s * ept + i * B) // CW, 8)
            pltpu.sync_copy(ends_hbm.at[pl.ds(rbase, R)], idx_m)
            for k in range(R):
                pltpu.sync_copy(ones_v, acc.at[idx_m.at[k]], add=True)
            return carry

        lax.fori_loop(0, nb, body, 0)
        plsc.subcore_barrier()

        @pl.when(s < wbt)
        def _():
            pltpu.sync_copy(acc.at[pl.ds(pl.multiple_of(s * wb, 8), wb)],
                            dbuf)
            pltpu.sync_copy(
                dbuf, out_hbm.at[pl.ds(pl.multiple_of(c * N + s * wb, 8), wb)])

    return deg_kernel


def _make_agg_kernel(N, EP, S):
    ppc = S // NC
    B = 1024
    R = B // CW
    ept = EP // NT
    nb = ept // B
    assert ept % B == 0
    wbt, wb = _wb_split(N)
    zr = 40
    wzr = 200
    assert wb % zr == 0 and wb % wzr == 0 and B >= wzr

    @functools.partial(
        pl.kernel,
        out_type=jax.ShapeDtypeStruct((S * N, DH), jnp.float32),
        mesh=_mesh(),
        scratch_types=[
            pltpu.VMEM((R, CW), jnp.int32),
            pltpu.VMEM((R, CW), jnp.int32),
            pltpu.VMEM((B, DH), jnp.float32),
            pltpu.VMEM((zr, DH), jnp.float32),
            pltpu.VMEM_SHARED((N + TR, DH), jnp.float32),
            pltpu.SemaphoreType.DMA,
        ],
        compiler_params=pltpu.CompilerParams(use_tc_tiling_on_sc=False),
    )
    def agg_kernel(x_hbm, gidx_hbm, dst_hbm, out_hbm,
                   gi_m, si_m, rows_v, zbuf, acc, sem):
        c = lax.axis_index("c")
        s = lax.axis_index("s")
        zero16 = jnp.zeros((LANE,), jnp.float32)
        for r in range(zr):
            for j in range(DH // LANE):
                zbuf[r, pl.ds(j * LANE, LANE)] = zero16

        erows = EP // CW
        etrows = ept // CW

        for p in range(ppc):
            q = c * ppc + p

            @pl.when(s < wbt)
            def _():
                def zloop(k, carry):
                    base = pl.multiple_of(s * wb + k * zr, 8)
                    pltpu.sync_copy(zbuf, acc.at[pl.ds(base, zr)])
                    return carry

                lax.fori_loop(0, wb // zr, zloop, 0)

            plsc.subcore_barrier()

            def body(i, carry):
                rbase = pl.multiple_of(s * etrows + i * R, 8)
                pltpu.sync_copy(
                    gidx_hbm.at[pl.ds(pl.multiple_of(q * erows + rbase, 8),
                                      R)], gi_m)
                pltpu.sync_copy(dst_hbm.at[pl.ds(rbase, R)], si_m)
                gets = [
                    pltpu.async_copy(x_hbm.at[gi_m.at[k]],
                                     rows_v.at[pl.ds(k * CW, CW)], sem)
                    for k in range(R)
                ]
                for k in range(R):
                    gets[k].wait()
                    pltpu.sync_copy(rows_v.at[pl.ds(k * CW, CW)],
                                    acc.at[si_m.at[k]], add=True)
                return carry

            lax.fori_loop(0, nb, body, 0)
            plsc.subcore_barrier()

            @pl.when(s < wbt)
            def _():
                def wloop(k, carry):
                    base = pl.multiple_of(s * wb + k * wzr, 8)
                    bb = rows_v.at[pl.ds(0, wzr)]
                    pltpu.sync_copy(acc.at[pl.ds(base, wzr)], bb)
                    obase = pl.multiple_of(q * N + base, 8)
                    pltpu.sync_copy(bb, out_hbm.at[pl.ds(obase, wzr)])
                    return carry

                lax.fori_loop(0, wb // wzr, wloop, 0)

            plsc.subcore_barrier()

    return agg_kernel


def _to_seg(x):
    return jnp.stack([x[:, q * DH:(q + 1) * DH]
                      for q in range(x.shape[-1] // DH)])


def _from_seg(a):
    return jnp.concatenate(list(a), axis=-1)


def _prep_body(deg_ref, feat_ref, inv_ref, x_ref):
    inv = lax.rsqrt(jnp.maximum(deg_ref[...], 1.0))
    inv_ref[...] = inv
    x_ref[...] = _to_seg(feat_ref[...] * inv[0])


def _l1_body(agg_ref, inv_ref, w_ref, b_ref, out_ref):
    ag = _from_seg(agg_ref[...])
    inv = inv_ref[...]
    h = jnp.dot(ag * inv[1], w_ref[...], preferred_element_type=jnp.float32)
    h = jnp.maximum(h + b_ref[...], 0.0)
    out_ref[...] = _to_seg(h * inv[0])


def _l2_body(agg_ref, inv_ref, w1_ref, b1_ref, w2_ref, out_ref):
    ag = _from_seg(agg_ref[...])
    inv = inv_ref[...]
    h = jnp.dot(ag * inv[1], w1_ref[...], preferred_element_type=jnp.float32)
    h = jnp.maximum(h + b1_ref[...], 0.0)
    z = jnp.dot(h * inv[0], w2_ref[...], preferred_element_type=jnp.float32)
    out_ref[...] = _to_seg(z)


def _l3_body(agg_ref, inv_ref, b_ref, out_ref):
    ag = _from_seg(agg_ref[...])
    out_ref[...] = ag * inv_ref[...][1] + b_ref[...]


def _row_spec(shape):
    if len(shape) == 3:
        return pl.BlockSpec(shape, lambda i: (0, i, 0))
    return pl.BlockSpec(shape, lambda i: (i, 0))


def _full_spec(shape):
    nd = len(shape)
    return pl.BlockSpec(shape, lambda i: (0,) * nd)


def kernel(features, edge_index, W0, b0, W1, b1, W2, b2):
    N, D = features.shape
    E = edge_index.shape[1]
    H1 = W0.shape[1]
    H2 = W1.shape[1]
    C = W2.shape[1]
    BN = 2000
    assert N % BN == 0 and D == 2 * DH and H1 == 4 * DH and C == 2 * DH
    grid = N // BN

    chunk = NT * 512
    EP = -(-E // chunk) * chunk
    npads = EP - E
    src = edge_index[0]
    dst = edge_index[1]
    spread = jnp.arange(npads, dtype=jnp.int32) % TR
    trash = N + spread
    dstp = jnp.concatenate([dst, trash])
    dst2d = dstp.reshape(EP // CW, CW)

    def gidx(S):
        parts = []
        for q in range(S):
            parts.append(src + q * N)
            parts.append(q * N + spread)
        return jnp.concatenate(parts).reshape(S * EP // CW, CW)

    gidx2 = gidx(2)
    gidx4 = gidx(4)
    ends2d = jnp.concatenate([src, trash, dst, trash]).reshape(
        2 * EP // CW, CW)

    deg = _make_deg_kernel(N, EP)(ends2d)
    degs = deg.reshape(2, N, 1)

    invs, x0 = pl.pallas_call(
        _prep_body,
        grid=(grid,),
        in_specs=[_row_spec((2, BN, 1)), _row_spec((BN, D))],
        out_specs=[_row_spec((2, BN, 1)), _row_spec((2, BN, DH))],
        out_shape=[jax.ShapeDtypeStruct((2, N, 1), jnp.float32),
                   jax.ShapeDtypeStruct((2, N, DH), jnp.float32)],
    )(degs, features)

    agg0 = _make_agg_kernel(N, EP, 2)(x0.reshape(2 * N, DH), gidx2, dst2d)

    x1 = pl.pallas_call(
        _l1_body,
        grid=(grid,),
        in_specs=[_row_spec((2, BN, DH)), _row_spec((2, BN, 1)),
                  _full_spec((D, H1)), _full_spec((1, H1))],
        out_specs=_row_spec((4, BN, DH)),
        out_shape=jax.ShapeDtypeStruct((4, N, DH), jnp.float32),
    )(agg0.reshape(2, N, DH), invs, W0, b0.reshape(1, H1))

    agg1 = _make_agg_kernel(N, EP, 4)(x1.reshape(4 * N, DH), gidx4, dst2d)

    z2 = pl.pallas_call(
        _l2_body,
        grid=(grid,),
        in_specs=[_row_spec((4, BN, DH)), _row_spec((2, BN, 1)),
                  _full_spec((H1, H2)), _full_spec((1, H2)),
                  _full_spec((H2, C))],
        out_specs=_row_spec((2, BN, DH)),
        out_shape=jax.ShapeDtypeStruct((2, N, DH), jnp.float32),
    )(agg1.reshape(4, N, DH), invs, W1, b1.reshape(1, H2), W2)

    agg2 = _make_agg_kernel(N, EP, 2)(z2.reshape(2 * N, DH), gidx2, dst2d)

    out = pl.pallas_call(
        _l3_body,
        grid=(grid,),
        in_specs=[_row_spec((2, BN, DH)), _row_spec((2, BN, 1)),
                  _full_spec((1, C))],
        out_specs=_row_spec((BN, C)),
        out_shape=jax.ShapeDtypeStruct((N, C), jnp.float32),
    )(agg2.reshape(2, N, DH), invs, b2.reshape(1, C))

    return out


# speedup vs baseline: 9.7373x; 1.0741x over previous
import functools

import jax
import jax.numpy as jnp
from jax import lax
from jax.experimental import pallas as pl
from jax.experimental.pallas import tpu as pltpu
from jax.experimental.pallas import tpu_sc as plsc

NC = 2
NT = 16
LANE = 16
DH = 64
CW = 128
TR = 64


def _mesh():
    return plsc.VectorSubcoreMesh(
        core_axis_name="c", subcore_axis_name="s", num_cores=NC,
        num_subcores=NT)


def _wb_split(N):
    wbt = next(t for t in range(NT, 0, -1) if N % t == 0 and (N // t) % 8 == 0)
    return wbt, N // wbt


def _make_deg_kernel(N, EP):
    B = 4096
    R = B // CW
    ept = EP // NT
    nb = ept // B
    assert ept % B == 0
    slab = ((-(-N // NT) + LANE - 1) // LANE) * LANE
    npad = slab * NT
    assert npad >= N + TR
    wbt, wb = _wb_split(N)

    @functools.partial(
        pl.kernel,
        out_type=jax.ShapeDtypeStruct((2 * N,), jnp.float32),
        mesh=_mesh(),
        scratch_types=[
            pltpu.VMEM((R, CW), jnp.int32),
            pltpu.VMEM((slab,), jnp.float32),
            pltpu.VMEM((CW,), jnp.float32),
            pltpu.VMEM((wb,), jnp.float32),
            pltpu.VMEM_SHARED((npad,), jnp.float32),
        ],
    )
    def deg_kernel(ends_hbm, out_hbm, idx_m, zbuf, ones_v, dbuf, acc):
        c = lax.axis_index("c")
        s = lax.axis_index("s")
        zero16 = jnp.zeros((LANE,), jnp.float32)
        one16 = jnp.ones((LANE,), jnp.float32)
        for j in range(slab // LANE):
            zbuf[pl.ds(j * LANE, LANE)] = zero16
        for j in range(CW // LANE):
            ones_v[pl.ds(j * LANE, LANE)] = one16
        pltpu.sync_copy(zbuf, acc.at[pl.ds(s * slab, slab)])
        plsc.subcore_barrier()

        def body(i, carry):
            rbase = pl.multiple_of((c * EP + s * ept + i * B) // CW, 8)
            pltpu.sync_copy(ends_hbm.at[pl.ds(rbase, R)], idx_m)
            for k in range(R):
                pltpu.sync_copy(ones_v, acc.at[idx_m.at[k]], add=True)
            return carry

        lax.fori_loop(0, nb, body, 0)
        plsc.subcore_barrier()

        @pl.when(s < wbt)
        def _():
            pltpu.sync_copy(acc.at[pl.ds(pl.multiple_of(s * wb, 8), wb)],
                            dbuf)
            pltpu.sync_copy(
                dbuf, out_hbm.at[pl.ds(pl.multiple_of(c * N + s * wb, 8), wb)])

    return deg_kernel


def _make_agg_kernel(N, EP, S):
    ppc = S // NC
    B = 1024
    R = B // CW
    ept = EP // NT
    nb = ept // B
    assert ept % B == 0
    wbt, wb = _wb_split(N)
    zr = 40
    wzr = 200
    assert wb % zr == 0 and wb % wzr == 0 and B >= wzr

    @functools.partial(
        pl.kernel,
        out_type=jax.ShapeDtypeStruct((S * N, DH), jnp.float32),
        mesh=_mesh(),
        scratch_types=[
            pltpu.VMEM((R, CW), jnp.int32),
            pltpu.VMEM((R, CW), jnp.int32),
            pltpu.VMEM((B, DH), jnp.float32),
            pltpu.VMEM((zr, DH), jnp.float32),
            pltpu.VMEM_SHARED((N + TR, DH), jnp.float32),
            pltpu.SemaphoreType.DMA,
            pltpu.SemaphoreType.DMA,
            pltpu.SemaphoreType.DMA,
        ],
        compiler_params=pltpu.CompilerParams(use_tc_tiling_on_sc=False),
    )
    def agg_kernel(x_hbm, gidx_hbm, dst_hbm, out_hbm,
                   gi_m, si_m, rows_v, zbuf, acc, sem, sem2, isem):
        c = lax.axis_index("c")
        s = lax.axis_index("s")
        zero16 = jnp.zeros((LANE,), jnp.float32)
        for r in range(zr):
            for j in range(DH // LANE):
                zbuf[r, pl.ds(j * LANE, LANE)] = zero16

        erows = EP // CW
        etrows = ept // CW

        for p in range(ppc):
            q = c * ppc + p

            @pl.when(s < wbt)
            def _():
                zs = [
                    pltpu.async_copy(
                        zbuf,
                        acc.at[pl.ds(pl.multiple_of(s * wb + k * zr, 8), zr)],
                        isem)
                    for k in range(wb // zr)
                ]
                for z in zs:
                    z.wait()

            plsc.subcore_barrier()

            def body(i, carry):
                rbase = pl.multiple_of(s * etrows + i * R, 8)
                ia = pltpu.async_copy(
                    gidx_hbm.at[pl.ds(pl.multiple_of(q * erows + rbase, 8),
                                      R)], gi_m, isem)
                ib = pltpu.async_copy(dst_hbm.at[pl.ds(rbase, R)], si_m, isem)
                ia.wait()
                ib.wait()
                gets = [
                    pltpu.async_copy(x_hbm.at[gi_m.at[k]],
                                     rows_v.at[pl.ds(k * CW, CW)], sem)
                    for k in range(R)
                ]
                puts = []
                for k in range(R):
                    gets[k].wait()
                    puts.append(
                        pltpu.async_copy(rows_v.at[pl.ds(k * CW, CW)],
                                         acc.at[si_m.at[k]], sem2, add=True))
                for pp in puts:
                    pp.wait()
                return carry

            lax.fori_loop(0, nb, body, 0)
            plsc.subcore_barrier()

            @pl.when(s < wbt)
            def _():
                nw = wb // wzr
                rds = [
                    pltpu.async_copy(
                        acc.at[pl.ds(pl.multiple_of(s * wb + k * wzr, 8),
                                     wzr)],
                        rows_v.at[pl.ds(k * wzr, wzr)], isem)
                    for k in range(nw)
                ]
                for r in rds:
                    r.wait()
                wrs = [
                    pltpu.async_copy(
                        rows_v.at[pl.ds(k * wzr, wzr)],
                        out_hbm.at[pl.ds(
                            pl.multiple_of(q * N + s * wb + k * wzr, 8),
                            wzr)], isem)
                    for k in range(nw)
                ]
                for w in wrs:
                    w.wait()

            plsc.subcore_barrier()

    return agg_kernel


def _to_seg(x):
    return jnp.stack([x[:, q * DH:(q + 1) * DH]
                      for q in range(x.shape[-1] // DH)])


def _from_seg(a):
    return jnp.concatenate(list(a), axis=-1)


def _prep_body(deg_ref, feat_ref, inv_ref, x_ref):
    inv = lax.rsqrt(jnp.maximum(deg_ref[...], 1.0))
    inv_ref[...] = inv
    x_ref[...] = _to_seg(feat_ref[...] * inv[0])


def _l1_body(agg_ref, inv_ref, w_ref, b_ref, out_ref):
    ag = _from_seg(agg_ref[...])
    inv = inv_ref[...]
    h = jnp.dot(ag * inv[1], w_ref[...], preferred_element_type=jnp.float32)
    h = jnp.maximum(h + b_ref[...], 0.0)
    out_ref[...] = _to_seg(h * inv[0])


def _l2_body(agg_ref, inv_ref, w1_ref, b1_ref, w2_ref, out_ref):
    ag = _from_seg(agg_ref[...])
    inv = inv_ref[...]
    h = jnp.dot(ag * inv[1], w1_ref[...], preferred_element_type=jnp.float32)
    h = jnp.maximum(h + b1_ref[...], 0.0)
    z = jnp.dot(h * inv[0], w2_ref[...], preferred_element_type=jnp.float32)
    out_ref[...] = _to_seg(z)


def _l3_body(agg_ref, inv_ref, b_ref, out_ref):
    ag = _from_seg(agg_ref[...])
    out_ref[...] = ag * inv_ref[...][1] + b_ref[...]


def _row_spec(shape):
    if len(shape) == 3:
        return pl.BlockSpec(shape, lambda i: (0, i, 0))
    return pl.BlockSpec(shape, lambda i: (i, 0))


def _full_spec(shape):
    nd = len(shape)
    return pl.BlockSpec(shape, lambda i: (0,) * nd)


def kernel(features, edge_index, W0, b0, W1, b1, W2, b2):
    N, D = features.shape
    E = edge_index.shape[1]
    H1 = W0.shape[1]
    H2 = W1.shape[1]
    C = W2.shape[1]
    BN = 2000
    assert N % BN == 0 and D == 2 * DH and H1 == 4 * DH and C == 2 * DH
    grid = N // BN

    chunk = NT * 512
    EP = -(-E // chunk) * chunk
    npads = EP - E
    src = edge_index[0]
    dst = edge_index[1]
    spread = jnp.arange(npads, dtype=jnp.int32) % TR
    trash = N + spread
    dstp = jnp.concatenate([dst, trash])
    dst2d = dstp.reshape(EP // CW, CW)

    def gidx(S):
        parts = []
        for q in range(S):
            parts.append(src + q * N)
            parts.append(q * N + spread)
        return jnp.concatenate(parts).reshape(S * EP // CW, CW)

    gidx2 = gidx(2)
    gidx4 = gidx(4)
    ends2d = jnp.concatenate([src, trash, dst, trash]).reshape(
        2 * EP // CW, CW)

    deg = _make_deg_kernel(N, EP)(ends2d)
    degs = deg.reshape(2, N, 1)

    invs, x0 = pl.pallas_call(
        _prep_body,
        grid=(grid,),
        in_specs=[_row_spec((2, BN, 1)), _row_spec((BN, D))],
        out_specs=[_row_spec((2, BN, 1)), _row_spec((2, BN, DH))],
        out_shape=[jax.ShapeDtypeStruct((2, N, 1), jnp.float32),
                   jax.ShapeDtypeStruct((2, N, DH), jnp.float32)],
    )(degs, features)

    agg0 = _make_agg_kernel(N, EP, 2)(x0.reshape(2 * N, DH), gidx2, dst2d)

    x1 = pl.pallas_call(
        _l1_body,
        grid=(grid,),
        in_specs=[_row_spec((2, BN, DH)), _row_spec((2, BN, 1)),
                  _full_spec((D, H1)), _full_spec((1, H1))],
        out_specs=_row_spec((4, BN, DH)),
        out_shape=jax.ShapeDtypeStruct((4, N, DH), jnp.float32),
    )(agg0.reshape(2, N, DH), invs, W0, b0.reshape(1, H1))

    agg1 = _make_agg_kernel(N, EP, 4)(x1.reshape(4 * N, DH), gidx4, dst2d)

    z2 = pl.pallas_call(
        _l2_body,
        grid=(grid,),
        in_specs=[_row_spec((4, BN, DH)), _row_spec((2, BN, 1)),
                  _full_spec((H1, H2)), _full_spec((1, H2)),
                  _full_spec((H2, C))],
        out_specs=_row_spec((2, BN, DH)),
        out_shape=jax.ShapeDtypeStruct((2, N, DH), jnp.float32),
    )(agg1.reshape(4, N, DH), invs, W1, b1.reshape(1, H2), W2)

    agg2 = _make_agg_kernel(N, EP, 2)(z2.reshape(2 * N, DH), gidx2, dst2d)

    out = pl.pallas_call(
        _l3_body,
        grid=(grid,),
        in_specs=[_row_spec((2, BN, DH)), _row_spec((2, BN, 1)),
                  _full_spec((1, C))],
        out_specs=_row_spec((BN, C)),
        out_shape=jax.ShapeDtypeStruct((N, C), jnp.float32),
    )(agg2.reshape(2, N, DH), invs, b2.reshape(1, C))

    return out


# speedup vs baseline: 10.0652x; 1.0337x over previous
import functools

import jax
import jax.numpy as jnp
from jax import lax
from jax.experimental import pallas as pl
from jax.experimental.pallas import tpu as pltpu
from jax.experimental.pallas import tpu_sc as plsc

NC = 2
NT = 16
LANE = 16
DH = 64
CW = 128
TR = 64


def _mesh():
    return plsc.VectorSubcoreMesh(
        core_axis_name="c", subcore_axis_name="s", num_cores=NC,
        num_subcores=NT)


def _wb_split(N):
    wbt = next(t for t in range(NT, 0, -1) if N % t == 0 and (N // t) % 8 == 0)
    return wbt, N // wbt


def _make_deg_kernel(N, EP):
    B = 4096
    R = B // CW
    ept = EP // NT
    nb = ept // B
    assert ept % B == 0
    slab = ((-(-N // NT) + LANE - 1) // LANE) * LANE
    npad = slab * NT
    assert npad >= N + TR
    wbt, wb = _wb_split(N)

    @functools.partial(
        pl.kernel,
        out_type=jax.ShapeDtypeStruct((2 * N,), jnp.float32),
        mesh=_mesh(),
        scratch_types=[
            pltpu.VMEM((R, CW), jnp.int32),
            pltpu.VMEM((slab,), jnp.float32),
            pltpu.VMEM((CW,), jnp.float32),
            pltpu.VMEM((wb,), jnp.float32),
            pltpu.VMEM_SHARED((npad,), jnp.float32),
        ],
    )
    def deg_kernel(ends_hbm, out_hbm, idx_m, zbuf, ones_v, dbuf, acc):
        c = lax.axis_index("c")
        s = lax.axis_index("s")
        zero16 = jnp.zeros((LANE,), jnp.float32)
        one16 = jnp.ones((LANE,), jnp.float32)
        for j in range(slab // LANE):
            zbuf[pl.ds(j * LANE, LANE)] = zero16
        for j in range(CW // LANE):
            ones_v[pl.ds(j * LANE, LANE)] = one16
        pltpu.sync_copy(zbuf, acc.at[pl.ds(s * slab, slab)])
        plsc.subcore_barrier()

        def body(i, carry):
            rbase = pl.multiple_of((c * EP + s * ept + i * B) // CW, 8)
            pltpu.sync_copy(ends_hbm.at[pl.ds(rbase, R)], idx_m)
            for k in range(R):
                pltpu.sync_copy(ones_v, acc.at[idx_m.at[k]], add=True)
            return carry

        lax.fori_loop(0, nb, body, 0)
        plsc.subcore_barrier()

        @pl.when(s < wbt)
        def _():
            pltpu.sync_copy(acc.at[pl.ds(pl.multiple_of(s * wb, 8), wb)],
                            dbuf)
            pltpu.sync_copy(
                dbuf, out_hbm.at[pl.ds(pl.multiple_of(c * N + s * wb, 8), wb)])

    return deg_kernel


def _make_agg_kernel(N, EP, S):
    ppc = S // NC
    B = 1024
    R = B // CW
    ept = EP // NT
    nb = ept // B
    assert ept % B == 0
    wbt, wb = _wb_split(N)
    zr = 40
    wzr = 200
    assert wb % zr == 0 and wb % wzr == 0 and B >= wzr

    @functools.partial(
        pl.kernel,
        out_type=jax.ShapeDtypeStruct((S * N, DH), jnp.float32),
        mesh=_mesh(),
        scratch_types=[
            pltpu.VMEM((R, CW), jnp.int32),
            pltpu.VMEM((R, CW), jnp.int32),
            pltpu.VMEM((R, CW), jnp.int32),
            pltpu.VMEM((R, CW), jnp.int32),
            pltpu.VMEM((B, DH), jnp.float32),
            pltpu.VMEM((zr, DH), jnp.float32),
            pltpu.VMEM_SHARED((N + TR, DH), jnp.float32),
            pltpu.SemaphoreType.DMA,
            pltpu.SemaphoreType.DMA,
            pltpu.SemaphoreType.DMA,
        ],
        compiler_params=pltpu.CompilerParams(use_tc_tiling_on_sc=False),
    )
    def agg_kernel(x_hbm, gidx_hbm, dst_hbm, out_hbm,
                   gi0, si0, gi1, si1, rows_v, zbuf, acc, sem, sem2, isem):
        c = lax.axis_index("c")
        s = lax.axis_index("s")
        zero16 = jnp.zeros((LANE,), jnp.float32)
        for r in range(zr):
            for j in range(DH // LANE):
                zbuf[r, pl.ds(j * LANE, LANE)] = zero16

        erows = EP // CW
        etrows = ept // CW

        for p in range(ppc):
            q = c * ppc + p

            @pl.when(s < wbt)
            def _():
                zs = [
                    pltpu.async_copy(
                        zbuf,
                        acc.at[pl.ds(pl.multiple_of(s * wb + k * zr, 8), zr)],
                        isem)
                    for k in range(wb // zr)
                ]
                for z in zs:
                    z.wait()

            plsc.subcore_barrier()

            def stage(bi, gi, si):
                rbase = pl.multiple_of(s * etrows + bi * R, 8)
                ia = pltpu.async_copy(
                    gidx_hbm.at[pl.ds(pl.multiple_of(q * erows + rbase, 8),
                                      R)], gi, isem)
                ib = pltpu.async_copy(dst_hbm.at[pl.ds(rbase, R)], si, isem)
                return ia, ib

            def process(gi, si):
                gets = [
                    pltpu.async_copy(x_hbm.at[gi.at[k]],
                                     rows_v.at[pl.ds(k * CW, CW)], sem)
                    for k in range(R)
                ]
                puts = []
                for k in range(R):
                    gets[k].wait()
                    puts.append(
                        pltpu.async_copy(rows_v.at[pl.ds(k * CW, CW)],
                                         acc.at[si.at[k]], sem2, add=True))
                for pp in puts:
                    pp.wait()

            def body(i, carry):
                ia, ib = stage(2 * i, gi0, si0)
                ia.wait()
                ib.wait()
                pa, pb = stage(2 * i + 1, gi1, si1)
                process(gi0, si0)
                pa.wait()
                pb.wait()
                process(gi1, si1)
                return carry

            lax.fori_loop(0, nb // 2, body, 0)
            plsc.subcore_barrier()

            @pl.when(s < wbt)
            def _():
                nw = wb // wzr
                rds = [
                    pltpu.async_copy(
                        acc.at[pl.ds(pl.multiple_of(s * wb + k * wzr, 8),
                                     wzr)],
                        rows_v.at[pl.ds(k * wzr, wzr)], isem)
                    for k in range(nw)
                ]
                for r in rds:
                    r.wait()
                wrs = [
                    pltpu.async_copy(
                        rows_v.at[pl.ds(k * wzr, wzr)],
                        out_hbm.at[pl.ds(
                            pl.multiple_of(q * N + s * wb + k * wzr, 8),
                            wzr)], isem)
                    for k in range(nw)
                ]
                for w in wrs:
                    w.wait()

            plsc.subcore_barrier()

    return agg_kernel


def _to_seg(x):
    return jnp.stack([x[:, q * DH:(q + 1) * DH]
                      for q in range(x.shape[-1] // DH)])


def _from_seg(a):
    return jnp.concatenate(list(a), axis=-1)


def _prep_body(deg_ref, feat_ref, inv_ref, x_ref):
    inv = lax.rsqrt(jnp.maximum(deg_ref[...], 1.0))
    inv_ref[...] = inv
    x_ref[...] = _to_seg(feat_ref[...] * inv[0])


def _l1_body(agg_ref, inv_ref, w_ref, b_ref, out_ref):
    ag = _from_seg(agg_ref[...])
    inv = inv_ref[...]
    h = jnp.dot(ag * inv[1], w_ref[...], preferred_element_type=jnp.float32)
    h = jnp.maximum(h + b_ref[...], 0.0)
    out_ref[...] = _to_seg(h * inv[0])


def _l2_body(agg_ref, inv_ref, w1_ref, b1_ref, w2_ref, out_ref):
    ag = _from_seg(agg_ref[...])
    inv = inv_ref[...]
    h = jnp.dot(ag * inv[1], w1_ref[...], preferred_element_type=jnp.float32)
    h = jnp.maximum(h + b1_ref[...], 0.0)
    z = jnp.dot(h * inv[0], w2_ref[...], preferred_element_type=jnp.float32)
    out_ref[...] = _to_seg(z)


def _l3_body(agg_ref, inv_ref, b_ref, out_ref):
    ag = _from_seg(agg_ref[...])
    out_ref[...] = ag * inv_ref[...][1] + b_ref[...]


def _row_spec(shape):
    if len(shape) == 3:
        return pl.BlockSpec(shape, lambda i: (0, i, 0))
    return pl.BlockSpec(shape, lambda i: (i, 0))


def _full_spec(shape):
    nd = len(shape)
    return pl.BlockSpec(shape, lambda i: (0,) * nd)


def kernel(features, edge_index, W0, b0, W1, b1, W2, b2):
    N, D = features.shape
    E = edge_index.shape[1]
    H1 = W0.shape[1]
    H2 = W1.shape[1]
    C = W2.shape[1]
    BN = 2000
    assert N % BN == 0 and D == 2 * DH and H1 == 4 * DH and C == 2 * DH
    grid = N // BN

    chunk = NT * 512
    EP = -(-E // chunk) * chunk
    npads = EP - E
    src = edge_index[0]
    dst = edge_index[1]
    spread = jnp.arange(npads, dtype=jnp.int32) % TR
    trash = N + spread
    dstp = jnp.concatenate([dst, trash])
    dst2d = dstp.reshape(EP // CW, CW)

    def gidx(S):
        parts = []
        for q in range(S):
            parts.append(src + q * N)
            parts.append(q * N + spread)
        return jnp.concatenate(parts).reshape(S * EP // CW, CW)

    gidx2 = gidx(2)
    gidx4 = gidx(4)
    ends2d = jnp.concatenate([src, trash, dst, trash]).reshape(
        2 * EP // CW, CW)

    deg = _make_deg_kernel(N, EP)(ends2d)
    degs = deg.reshape(2, N, 1)

    invs, x0 = pl.pallas_call(
        _prep_body,
        grid=(grid,),
        in_specs=[_row_spec((2, BN, 1)), _row_spec((BN, D))],
        out_specs=[_row_spec((2, BN, 1)), _row_spec((2, BN, DH))],
        out_shape=[jax.ShapeDtypeStruct((2, N, 1), jnp.float32),
                   jax.ShapeDtypeStruct((2, N, DH), jnp.float32)],
    )(degs, features)

    agg0 = _make_agg_kernel(N, EP, 2)(x0.reshape(2 * N, DH), gidx2, dst2d)

    x1 = pl.pallas_call(
        _l1_body,
        grid=(grid,),
        in_specs=[_row_spec((2, BN, DH)), _row_spec((2, BN, 1)),
                  _full_spec((D, H1)), _full_spec((1, H1))],
        out_specs=_row_spec((4, BN, DH)),
        out_shape=jax.ShapeDtypeStruct((4, N, DH), jnp.float32),
    )(agg0.reshape(2, N, DH), invs, W0, b0.reshape(1, H1))

    agg1 = _make_agg_kernel(N, EP, 4)(x1.reshape(4 * N, DH), gidx4, dst2d)

    z2 = pl.pallas_call(
        _l2_body,
        grid=(grid,),
        in_specs=[_row_spec((4, BN, DH)), _row_spec((2, BN, 1)),
                  _full_spec((H1, H2)), _full_spec((1, H2)),
                  _full_spec((H2, C))],
        out_specs=_row_spec((2, BN, DH)),
        out_shape=jax.ShapeDtypeStruct((2, N, DH), jnp.float32),
    )(agg1.reshape(4, N, DH), invs, W1, b1.reshape(1, H2), W2)

    agg2 = _make_agg_kernel(N, EP, 2)(z2.reshape(2 * N, DH), gidx2, dst2d)

    out = pl.pallas_call(
        _l3_body,
        grid=(grid,),
        in_specs=[_row_spec((2, BN, DH)), _row_spec((2, BN, 1)),
                  _full_spec((1, C))],
        out_specs=_row_spec((BN, C)),
        out_shape=jax.ShapeDtypeStruct((N, C), jnp.float32),
    )(agg2.reshape(2, N, DH), invs, b2.reshape(1, C))

    return out


# speedup vs baseline: 10.2228x; 1.0157x over previous
import functools

import jax
import jax.numpy as jnp
from jax import lax
from jax.experimental import pallas as pl
from jax.experimental.pallas import tpu as pltpu
from jax.experimental.pallas import tpu_sc as plsc

NC = 2
NT = 16
LANE = 16
DH = 64
CW = 128
TR = 64


def _mesh():
    return plsc.VectorSubcoreMesh(
        core_axis_name="c", subcore_axis_name="s", num_cores=NC,
        num_subcores=NT)


def _wb_split(N):
    wbt = next(t for t in range(NT, 0, -1) if N % t == 0 and (N // t) % 8 == 0)
    return wbt, N // wbt


def _make_deg_kernel(N, EP):
    B = 4096
    R = B // CW
    ept = EP // NT
    nb = ept // B
    assert ept % B == 0
    slab = ((-(-N // NT) + LANE - 1) // LANE) * LANE
    npad = slab * NT
    assert npad >= N + TR
    wbt, wb = _wb_split(N)

    @functools.partial(
        pl.kernel,
        out_type=jax.ShapeDtypeStruct((2 * N,), jnp.float32),
        mesh=_mesh(),
        scratch_types=[
            pltpu.VMEM((R, CW), jnp.int32),
            pltpu.VMEM((slab,), jnp.float32),
            pltpu.VMEM((CW,), jnp.float32),
            pltpu.VMEM((wb,), jnp.float32),
            pltpu.VMEM_SHARED((npad,), jnp.float32),
            pltpu.SemaphoreType.DMA,
        ],
    )
    def deg_kernel(ends_hbm, out_hbm, idx_m, zbuf, ones_v, dbuf, acc, dsem):
        c = lax.axis_index("c")
        s = lax.axis_index("s")
        zero16 = jnp.zeros((LANE,), jnp.float32)
        one16 = jnp.ones((LANE,), jnp.float32)
        for j in range(slab // LANE):
            zbuf[pl.ds(j * LANE, LANE)] = zero16
        for j in range(CW // LANE):
            ones_v[pl.ds(j * LANE, LANE)] = one16
        pltpu.sync_copy(zbuf, acc.at[pl.ds(s * slab, slab)])
        plsc.subcore_barrier()

        def body(i, carry):
            rbase = pl.multiple_of((c * EP + s * ept + i * B) // CW, 8)
            pltpu.sync_copy(ends_hbm.at[pl.ds(rbase, R)], idx_m)
            puts = [
                pltpu.async_copy(ones_v, acc.at[idx_m.at[k]], dsem, add=True)
                for k in range(R)
            ]
            for pp in puts:
                pp.wait()
            return carry

        lax.fori_loop(0, nb, body, 0)
        plsc.subcore_barrier()

        @pl.when(s < wbt)
        def _():
            pltpu.sync_copy(acc.at[pl.ds(pl.multiple_of(s * wb, 8), wb)],
                            dbuf)
            pltpu.sync_copy(
                dbuf, out_hbm.at[pl.ds(pl.multiple_of(c * N + s * wb, 8), wb)])

    return deg_kernel


def _make_agg_kernel(N, EP, S):
    ppc = S // NC
    B = 1024
    R = B // CW
    ept = EP // NT
    nb = ept // B
    assert ept % B == 0
    wbt, wb = _wb_split(N)
    zr = 40
    wzr = 200
    assert wb % zr == 0 and wb % wzr == 0 and B >= wzr

    @functools.partial(
        pl.kernel,
        out_type=jax.ShapeDtypeStruct((S * N, DH), jnp.float32),
        mesh=_mesh(),
        scratch_types=[
            pltpu.VMEM((R, CW), jnp.int32),
            pltpu.VMEM((R, CW), jnp.int32),
            pltpu.VMEM((R, CW), jnp.int32),
            pltpu.VMEM((R, CW), jnp.int32),
            pltpu.VMEM((B, DH), jnp.float32),
            pltpu.VMEM((zr, DH), jnp.float32),
            pltpu.VMEM_SHARED((N + TR, DH), jnp.float32),
            pltpu.SemaphoreType.DMA,
            pltpu.SemaphoreType.DMA,
            pltpu.SemaphoreType.DMA,
        ],
        compiler_params=pltpu.CompilerParams(use_tc_tiling_on_sc=False),
    )
    def agg_kernel(x_hbm, gidx_hbm, dst_hbm, out_hbm,
                   gi0, si0, gi1, si1, rows_v, zbuf, acc, sem, sem2, isem):
        c = lax.axis_index("c")
        s = lax.axis_index("s")
        zero16 = jnp.zeros((LANE,), jnp.float32)
        for r in range(zr):
            for j in range(DH // LANE):
                zbuf[r, pl.ds(j * LANE, LANE)] = zero16

        erows = EP // CW
        etrows = ept // CW

        for p in range(ppc):
            q = c * ppc + p

            @pl.when(s < wbt)
            def _():
                zs = [
                    pltpu.async_copy(
                        zbuf,
                        acc.at[pl.ds(pl.multiple_of(s * wb + k * zr, 8), zr)],
                        isem)
                    for k in range(wb // zr)
                ]
                for z in zs:
                    z.wait()

            plsc.subcore_barrier()

            def stage(bi, gi, si):
                rbase = pl.multiple_of(s * etrows + bi * R, 8)
                ia = pltpu.async_copy(
                    gidx_hbm.at[pl.ds(pl.multiple_of(q * erows + rbase, 8),
                                      R)], gi, isem)
                ib = pltpu.async_copy(dst_hbm.at[pl.ds(rbase, R)], si, isem)
                return ia, ib

            def process(gi, si):
                gets = [
                    pltpu.async_copy(x_hbm.at[gi.at[k]],
                                     rows_v.at[pl.ds(k * CW, CW)], sem)
                    for k in range(R)
                ]
                puts = []
                for k in range(R):
                    gets[k].wait()
                    puts.append(
                        pltpu.async_copy(rows_v.at[pl.ds(k * CW, CW)],
                                         acc.at[si.at[k]], sem2, add=True))
                for pp in puts:
                    pp.wait()

            def body(i, carry):
                ia, ib = stage(2 * i, gi0, si0)
                ia.wait()
                ib.wait()
                pa, pb = stage(2 * i + 1, gi1, si1)
                process(gi0, si0)
                pa.wait()
                pb.wait()
                process(gi1, si1)
                return carry

            lax.fori_loop(0, nb // 2, body, 0)
            plsc.subcore_barrier()

            @pl.when(s < wbt)
            def _():
                nw = wb // wzr
                rds = [
                    pltpu.async_copy(
                        acc.at[pl.ds(pl.multiple_of(s * wb + k * wzr, 8),
                                     wzr)],
                        rows_v.at[pl.ds(k * wzr, wzr)], isem)
                    for k in range(nw)
                ]
                for r in rds:
                    r.wait()
                wrs = [
                    pltpu.async_copy(
                        rows_v.at[pl.ds(k * wzr, wzr)],
                        out_hbm.at[pl.ds(
                            pl.multiple_of(q * N + s * wb + k * wzr, 8),
                            wzr)], isem)
                    for k in range(nw)
                ]
                for w in wrs:
                    w.wait()

            plsc.subcore_barrier()

    return agg_kernel


def _to_seg(x):
    return jnp.stack([x[:, q * DH:(q + 1) * DH]
                      for q in range(x.shape[-1] // DH)])


def _from_seg(a):
    return jnp.concatenate(list(a), axis=-1)


def _prep_body(deg_ref, feat_ref, inv_ref, x_ref):
    inv = lax.rsqrt(jnp.maximum(deg_ref[...], 1.0))
    inv_ref[...] = inv
    x_ref[...] = _to_seg(feat_ref[...] * inv[0])


def _l1_body(agg_ref, inv_ref, w_ref, b_ref, out_ref):
    ag = _from_seg(agg_ref[...])
    inv = inv_ref[...]
    h = jnp.dot(ag * inv[1], w_ref[...], preferred_element_type=jnp.float32)
    h = jnp.maximum(h + b_ref[...], 0.0)
    out_ref[...] = _to_seg(h * inv[0])


def _l2_body(agg_ref, inv_ref, w1_ref, b1_ref, w2_ref, out_ref):
    ag = _from_seg(agg_ref[...])
    inv = inv_ref[...]
    h = jnp.dot(ag * inv[1], w1_ref[...], preferred_element_type=jnp.float32)
    h = jnp.maximum(h + b1_ref[...], 0.0)
    z = jnp.dot(h * inv[0], w2_ref[...], preferred_element_type=jnp.float32)
    out_ref[...] = _to_seg(z)


def _l3_body(agg_ref, inv_ref, b_ref, out_ref):
    ag = _from_seg(agg_ref[...])
    out_ref[...] = ag * inv_ref[...][1] + b_ref[...]


def _row_spec(shape):
    if len(shape) == 3:
        return pl.BlockSpec(shape, lambda i: (0, i, 0))
    return pl.BlockSpec(shape, lambda i: (i, 0))


def _full_spec(shape):
    nd = len(shape)
    return pl.BlockSpec(shape, lambda i: (0,) * nd)


def kernel(features, edge_index, W0, b0, W1, b1, W2, b2):
    N, D = features.shape
    E = edge_index.shape[1]
    H1 = W0.shape[1]
    H2 = W1.shape[1]
    C = W2.shape[1]
    BN = 2000
    assert N % BN == 0 and D == 2 * DH and H1 == 4 * DH and C == 2 * DH
    grid = N // BN

    chunk = NT * 512
    EP = -(-E // chunk) * chunk
    npads = EP - E
    src = edge_index[0]
    dst = edge_index[1]
    spread = jnp.arange(npads, dtype=jnp.int32) % TR
    trash = N + spread
    dstp = jnp.concatenate([dst, trash])
    dst2d = dstp.reshape(EP // CW, CW)

    def gidx(S):
        parts = []
        for q in range(S):
            parts.append(src + q * N)
            parts.append(q * N + spread)
        return jnp.concatenate(parts).reshape(S * EP // CW, CW)

    gidx2 = gidx(2)
    gidx4 = gidx(4)
    ends2d = jnp.concatenate([src, trash, dst, trash]).reshape(
        2 * EP // CW, CW)

    deg = _make_deg_kernel(N, EP)(ends2d)
    degs = deg.reshape(2, N, 1)

    invs, x0 = pl.pallas_call(
        _prep_body,
        grid=(grid,),
        in_specs=[_row_spec((2, BN, 1)), _row_spec((BN, D))],
        out_specs=[_row_spec((2, BN, 1)), _row_spec((2, BN, DH))],
        out_shape=[jax.ShapeDtypeStruct((2, N, 1), jnp.float32),
                   jax.ShapeDtypeStruct((2, N, DH), jnp.float32)],
    )(degs, features)

    agg0 = _make_agg_kernel(N, EP, 2)(x0.reshape(2 * N, DH), gidx2, dst2d)

    x1 = pl.pallas_call(
        _l1_body,
        grid=(grid,),
        in_specs=[_row_spec((2, BN, DH)), _row_spec((2, BN, 1)),
                  _full_spec((D, H1)), _full_spec((1, H1))],
        out_specs=_row_spec((4, BN, DH)),
        out_shape=jax.ShapeDtypeStruct((4, N, DH), jnp.float32),
    )(agg0.reshape(2, N, DH), invs, W0, b0.reshape(1, H1))

    agg1 = _make_agg_kernel(N, EP, 4)(x1.reshape(4 * N, DH), gidx4, dst2d)

    z2 = pl.pallas_call(
        _l2_body,
        grid=(grid,),
        in_specs=[_row_spec((4, BN, DH)), _row_spec((2, BN, 1)),
                  _full_spec((H1, H2)), _full_spec((1, H2)),
                  _full_spec((H2, C))],
        out_specs=_row_spec((2, BN, DH)),
        out_shape=jax.ShapeDtypeStruct((2, N, DH), jnp.float32),
    )(agg1.reshape(4, N, DH), invs, W1, b1.reshape(1, H2), W2)

    agg2 = _make_agg_kernel(N, EP, 2)(z2.reshape(2 * N, DH), gidx2, dst2d)

    out = pl.pallas_call(
        _l3_body,
        grid=(grid,),
        in_specs=[_row_spec((2, BN, DH)), _row_spec((2, BN, 1)),
                  _full_spec((1, C))],
        out_specs=_row_spec((BN, C)),
        out_shape=jax.ShapeDtypeStruct((N, C), jnp.float32),
    )(agg2.reshape(2, N, DH), invs, b2.reshape(1, C))

    return out
